# baseline (speedup 1.0000x reference)
import functools

import jax
import jax.numpy as jnp
from jax import lax
from jax.experimental import pallas as pl
from jax.experimental.pallas import tpu as pltpu
from jax.experimental.pallas import tpu_sc as plsc

_NEG = float(jnp.finfo(jnp.float32).min)

_BT = 512
_CH = 2048
_NFULL = 48
_TAIL = 1696
_UNROLL = 8


def _tc_body(y_ref, x_ref, o_ref, *, br, v):
    i = pl.program_id(0)
    x = x_ref[...]
    s = jnp.sum(jnp.exp(x), axis=1, keepdims=True)

    lane = lax.broadcasted_iota(jnp.int32, (1, 128), 1)
    seg_info = []
    for r in range(br):
        yr = y_ref[i * br + r]
        seg = (yr // 128) * 128
        xs = x_ref[r, pl.ds(seg, 128)].reshape(1, 128)
        seg_info.append((yr - seg, xs))
        x_ref[r, pl.ds(seg, 128)] = jnp.full((128,), _NEG, jnp.float32)

    m_excl = jnp.max(x_ref[...], axis=1, keepdims=True)
    seg_max = [
        jnp.max(jnp.where(lane == off, _NEG, xs)) for off, xs in seg_info
    ]
    first = jnp.maximum(m_excl, jnp.stack(seg_max).reshape(br, 1))
    o_ref[...] = (first - jnp.log(s - first)).reshape(1, 1, br)


def _sc_finalize_body(m1l_ref, m2l_ref, sl_ref, seg_ref, y_ref, o_ref):
    m1l = m1l_ref[...]
    m2l = m2l_ref[...]
    m1 = jnp.max(m1l, axis=1, keepdims=True)
    eq = m1l == m1
    cnt = jnp.sum(eq.astype(jnp.float32), axis=1, keepdims=True)
    mx2 = jnp.max(jnp.where(eq, _NEG, m1l), axis=1, keepdims=True)
    m2 = jnp.where(cnt > 1.0, m1, mx2)
    m2 = jnp.maximum(m2, jnp.max(m2l, axis=1, keepdims=True))
    s = jnp.sum(sl_ref[...], axis=1, keepdims=True)

    segs = seg_ref[...]
    offs = lax.rem(y_ref[...], 16)
    lane = lax.broadcasted_iota(jnp.int32, segs.shape, 1)
    vy = jnp.sum(jnp.where(lane == offs, segs, 0.0), axis=1, keepdims=True)

    first = jnp.where(vy >= m1, m2, m1)
    o_ref[...] = (first - jnp.log(s - first)).reshape(-1)


def _make_sc_stream(b, v, bt):
    bs = b - bt
    info = plsc.get_sparse_core_info()
    nw = info.num_cores * info.num_subcores
    n = bs // nw
    nch = v // _CH

    mesh = plsc.VectorSubcoreMesh(core_axis_name="c", subcore_axis_name="s")
    stat = jax.ShapeDtypeStruct((bs * 16,), jnp.float32)

    @functools.partial(
        pl.kernel,
        mesh=mesh,
        out_type=(stat, stat, stat, stat),
        scratch_types=[
            pltpu.VMEM((n,), jnp.int32),
            pltpu.VMEM((n * 16,), jnp.float32),
            pltpu.VMEM((8, _CH), jnp.float32),
            pltpu.VMEM((8, _CH), jnp.float32),
            pltpu.VMEM((8, _TAIL), jnp.float32),
            pltpu.VMEM((n * 16,), jnp.float32),
            pltpu.VMEM((n * 16,), jnp.float32),
            pltpu.VMEM((n * 16,), jnp.float32),
            pltpu.SemaphoreType.DMA,
            pltpu.SemaphoreType.DMA,
            pltpu.SemaphoreType.DMA,
        ],
    )
    def sc_stream(
        x_hbm, y_hbm, m1_hbm, m2_hbm, s_hbm, seg_hbm,
        y_v, segv, bufa, bufb, buft, m1v, m2v, sv, semg, sema, semb,
    ):
        wid = lax.axis_index("s") * info.num_cores + lax.axis_index("c")
        base = wid * n
        pltpu.sync_copy(y_hbm.at[pl.ds(bt + base, n)], y_v)

        copies = []
        for g in range(n // 16):
            yvec = y_v[pl.ds(g * 16, 16)]
            for j in range(16):
                r = g * 16 + j
                seg = (yvec[j] // 16) * 16
                copies.append(
                    pltpu.async_copy(
                        x_hbm.at[bt + base + r, pl.ds(seg, 16)],
                        segv.at[pl.ds(r * 16, 16)],
                        sem=semg,
                    )
                )
        for cp in copies:
            cp.wait()
        pltpu.sync_copy(segv, seg_hbm.at[pl.ds(base * 16, n * 16)])

        def consume(buf, accs, nslices, unroll):
            new = []
            for r in range(8):
                def sl_body(k, carry, r=r, buf=buf, unroll=unroll):
                    a1, a2, cs = carry
                    for u in range(unroll):
                        vx = buf[r, pl.ds((k * unroll + u) * 16, 16)]
                        cs = cs + jnp.exp(vx)
                        a2 = jnp.maximum(a2, jnp.minimum(a1, vx))
                        a1 = jnp.maximum(a1, vx)
                    return a1, a2, cs

                new.append(
                    lax.fori_loop(0, nslices // unroll, sl_body, accs[r])
                )
            return new

        def slab_body(slab, _):
            srow = pl.multiple_of(bt + base + slab * 8, 8)
            zero = jnp.zeros((16,), jnp.float32)
            neg = jnp.full((16,), _NEG, jnp.float32)
            accs = [(neg, neg, zero) for _ in range(8)]

            pltpu.async_copy(
                x_hbm.at[pl.ds(srow, 8), pl.ds(0, _CH)], bufa, sem=sema
            )
            pltpu.async_copy(
                x_hbm.at[pl.ds(srow, 8), pl.ds(_CH, _CH)], bufb, sem=semb
            )

            def pair_body(p, carry, srow=srow):
                accs = [tuple(carry[r * 3 + k] for k in range(3))
                        for r in range(8)]
                pltpu.make_async_copy(
                    x_hbm.at[pl.ds(srow, 8), pl.ds(0, _CH)], bufa, sema
                ).wait()
                accs = consume(bufa, accs, _CH // 16, _UNROLL)

                @pl.when(p < _NFULL // 2 - 1)
                def _():
                    pltpu.async_copy(
                        x_hbm.at[pl.ds(srow, 8), pl.ds((2 * p + 2) * _CH, _CH)],
                        bufa,
                        sem=sema,
                    )

                pltpu.make_async_copy(
                    x_hbm.at[pl.ds(srow, 8), pl.ds(0, _CH)], bufb, semb
                ).wait()
                accs = consume(bufb, accs, _CH // 16, _UNROLL)

                @pl.when(p < _NFULL // 2 - 1)
                def _():
                    pltpu.async_copy(
                        x_hbm.at[pl.ds(srow, 8), pl.ds((2 * p + 3) * _CH, _CH)],
                        bufb,
                        sem=semb,
                    )

                return tuple(accs[r][k] for r in range(8) for k in range(3))

            flat = tuple(accs[r][k] for r in range(8) for k in range(3))
            flat = lax.fori_loop(0, _NFULL // 2, pair_body, flat)
            accs = [tuple(flat[r * 3 + k] for k in range(3)) for r in range(8)]

            pltpu.sync_copy(
                x_hbm.at[pl.ds(srow, 8), pl.ds(_NFULL * _CH, _TAIL)], buft
            )
            accs = consume(buft, accs, _TAIL // 16, 2)

            for r in range(8):
                t = slab * 8 + r
                m1v[pl.ds(t * 16, 16)] = accs[r][0]
                m2v[pl.ds(t * 16, 16)] = accs[r][1]
                sv[pl.ds(t * 16, 16)] = accs[r][2]
            return 0

        lax.fori_loop(0, n // 8, slab_body, 0)

        pltpu.sync_copy(m1v, m1_hbm.at[pl.ds(base * 16, n * 16)])
        pltpu.sync_copy(m2v, m2_hbm.at[pl.ds(base * 16, n * 16)])
        pltpu.sync_copy(sv, s_hbm.at[pl.ds(base * 16, n * 16)])

    return sc_stream


def kernel(logits, y):
    b, v = logits.shape
    bt = _BT
    bs = b - bt
    br = 32
    grid = bt // br
    y32 = y.astype(jnp.int32)

    if bt > 0:
        out_tc = pl.pallas_call(
            functools.partial(_tc_body, br=br, v=v),
            grid=(grid,),
            in_specs=[
                pl.BlockSpec(memory_space=pltpu.SMEM),
                pl.BlockSpec((br, v), lambda i: (i, 0)),
            ],
            out_specs=pl.BlockSpec((1, 1, br), lambda i: (i, 0, 0)),
            out_shape=jax.ShapeDtypeStruct((grid, 1, br), jnp.float32),
        )(y32, logits)

    m1l, m2l, sl, segs = _make_sc_stream(b, v, bt)(logits, y32)

    out_sc = pl.pallas_call(
        _sc_finalize_body, out_shape=jax.ShapeDtypeStruct((bs,), jnp.float32)
    )(
        m1l.reshape(bs, 16),
        m2l.reshape(bs, 16),
        sl.reshape(bs, 16),
        segs.reshape(bs, 16),
        y32[bt:].reshape(bs, 1),
    )
    if bt == 0:
        return out_sc
    return jnp.concatenate([out_tc.reshape(bt), out_sc])


# speedup vs baseline: 1.0228x; 1.0228x over previous
import functools

import jax
import jax.numpy as jnp
from jax import lax
from jax.experimental import pallas as pl
from jax.experimental.pallas import tpu as pltpu
from jax.experimental.pallas import tpu_sc as plsc

_NEG = float(jnp.finfo(jnp.float32).min)

_BT = 512
_CH = 4096
_NFULL = 24
_TAIL = 1696
_UNROLL = 8


def _tc_body(y_ref, x_ref, o_ref, *, br, v):
    i = pl.program_id(0)
    x = x_ref[...]
    s = jnp.sum(jnp.exp(x), axis=1, keepdims=True)

    lane = lax.broadcasted_iota(jnp.int32, (1, 128), 1)
    seg_info = []
    for r in range(br):
        yr = y_ref[i * br + r]
        seg = (yr // 128) * 128
        xs = x_ref[r, pl.ds(seg, 128)].reshape(1, 128)
        seg_info.append((yr - seg, xs))
        x_ref[r, pl.ds(seg, 128)] = jnp.full((128,), _NEG, jnp.float32)

    m_excl = jnp.max(x_ref[...], axis=1, keepdims=True)
    seg_max = [
        jnp.max(jnp.where(lane == off, _NEG, xs)) for off, xs in seg_info
    ]
    first = jnp.maximum(m_excl, jnp.stack(seg_max).reshape(br, 1))
    o_ref[...] = (first - jnp.log(s - first)).reshape(1, 1, br)


def _sc_finalize_body(m1l_ref, m2l_ref, sl_ref, seg_ref, y_ref, o_ref):
    m1l = m1l_ref[...]
    m2l = m2l_ref[...]
    m1 = jnp.max(m1l, axis=1, keepdims=True)
    eq = m1l == m1
    cnt = jnp.sum(eq.astype(jnp.float32), axis=1, keepdims=True)
    mx2 = jnp.max(jnp.where(eq, _NEG, m1l), axis=1, keepdims=True)
    m2 = jnp.where(cnt > 1.0, m1, mx2)
    m2 = jnp.maximum(m2, jnp.max(m2l, axis=1, keepdims=True))
    s = jnp.sum(sl_ref[...], axis=1, keepdims=True)

    segs = seg_ref[...]
    offs = lax.rem(y_ref[...], 16)
    lane = lax.broadcasted_iota(jnp.int32, segs.shape, 1)
    vy = jnp.sum(jnp.where(lane == offs, segs, 0.0), axis=1, keepdims=True)

    first = jnp.where(vy >= m1, m2, m1)
    o_ref[...] = (first - jnp.log(s - first)).reshape(-1)


def _make_sc_stream(b, v, bt):
    bs = b - bt
    info = plsc.get_sparse_core_info()
    nw = info.num_cores * info.num_subcores
    n = bs // nw
    nch = v // _CH

    mesh = plsc.VectorSubcoreMesh(core_axis_name="c", subcore_axis_name="s")
    stat = jax.ShapeDtypeStruct((bs * 16,), jnp.float32)

    @functools.partial(
        pl.kernel,
        mesh=mesh,
        out_type=(stat, stat, stat, stat),
        scratch_types=[
            pltpu.VMEM((n,), jnp.int32),
            pltpu.VMEM((n * 16,), jnp.float32),
            pltpu.VMEM((8, _CH), jnp.float32),
            pltpu.VMEM((8, _CH), jnp.float32),
            pltpu.VMEM((8, _TAIL), jnp.float32),
            pltpu.VMEM((n * 16,), jnp.float32),
            pltpu.VMEM((n * 16,), jnp.float32),
            pltpu.VMEM((n * 16,), jnp.float32),
            pltpu.SemaphoreType.DMA,
            pltpu.SemaphoreType.DMA,
            pltpu.SemaphoreType.DMA,
        ],
    )
    def sc_stream(
        x_hbm, y_hbm, m1_hbm, m2_hbm, s_hbm, seg_hbm,
        y_v, segv, bufa, bufb, buft, m1v, m2v, sv, semg, sema, semb,
    ):
        wid = lax.axis_index("s") * info.num_cores + lax.axis_index("c")
        base = wid * n
        pltpu.sync_copy(y_hbm.at[pl.ds(bt + base, n)], y_v)

        copies = []
        for g in range(n // 16):
            yvec = y_v[pl.ds(g * 16, 16)]
            for j in range(16):
                r = g * 16 + j
                seg = (yvec[j] // 16) * 16
                copies.append(
                    pltpu.async_copy(
                        x_hbm.at[bt + base + r, pl.ds(seg, 16)],
                        segv.at[pl.ds(r * 16, 16)],
                        sem=semg,
                    )
                )
        for cp in copies:
            cp.wait()
        pltpu.sync_copy(segv, seg_hbm.at[pl.ds(base * 16, n * 16)])

        def consume(buf, accs, nslices, unroll):
            new = []
            for r in range(8):
                def sl_body(k, carry, r=r, buf=buf, unroll=unroll):
                    a1, a2, cs = carry
                    for u in range(unroll):
                        vx = buf[r, pl.ds((k * unroll + u) * 16, 16)]
                        cs = cs + jnp.exp(vx)
                        a2 = jnp.maximum(a2, jnp.minimum(a1, vx))
                        a1 = jnp.maximum(a1, vx)
                    return a1, a2, cs

                new.append(
                    lax.fori_loop(0, nslices // unroll, sl_body, accs[r])
                )
            return new

        def slab_body(slab, _):
            srow = pl.multiple_of(bt + base + slab * 8, 8)
            zero = jnp.zeros((16,), jnp.float32)
            neg = jnp.full((16,), _NEG, jnp.float32)
            accs = [(neg, neg, zero) for _ in range(8)]

            pltpu.async_copy(
                x_hbm.at[pl.ds(srow, 8), pl.ds(0, _CH)], bufa, sem=sema
            )
            pltpu.async_copy(
                x_hbm.at[pl.ds(srow, 8), pl.ds(_CH, _CH)], bufb, sem=semb
            )

            def pair_body(p, carry, srow=srow):
                accs = [tuple(carry[r * 3 + k] for k in range(3))
                        for r in range(8)]
                pltpu.make_async_copy(
                    x_hbm.at[pl.ds(srow, 8), pl.ds(0, _CH)], bufa, sema
                ).wait()
                accs = consume(bufa, accs, _CH // 16, _UNROLL)

                @pl.when(p < _NFULL // 2 - 1)
                def _():
                    pltpu.async_copy(
                        x_hbm.at[pl.ds(srow, 8), pl.ds((2 * p + 2) * _CH, _CH)],
                        bufa,
                        sem=sema,
                    )

                pltpu.make_async_copy(
                    x_hbm.at[pl.ds(srow, 8), pl.ds(0, _CH)], bufb, semb
                ).wait()
                accs = consume(bufb, accs, _CH // 16, _UNROLL)

                @pl.when(p < _NFULL // 2 - 1)
                def _():
                    pltpu.async_copy(
                        x_hbm.at[pl.ds(srow, 8), pl.ds((2 * p + 3) * _CH, _CH)],
                        bufb,
                        sem=semb,
                    )

                return tuple(accs[r][k] for r in range(8) for k in range(3))

            flat = tuple(accs[r][k] for r in range(8) for k in range(3))
            flat = lax.fori_loop(0, _NFULL // 2, pair_body, flat)
            accs = [tuple(flat[r * 3 + k] for k in range(3)) for r in range(8)]

            pltpu.sync_copy(
                x_hbm.at[pl.ds(srow, 8), pl.ds(_NFULL * _CH, _TAIL)], buft
            )
            accs = consume(buft, accs, _TAIL // 16, 2)

            for r in range(8):
                t = slab * 8 + r
                m1v[pl.ds(t * 16, 16)] = accs[r][0]
                m2v[pl.ds(t * 16, 16)] = accs[r][1]
                sv[pl.ds(t * 16, 16)] = accs[r][2]
            return 0

        lax.fori_loop(0, n // 8, slab_body, 0)

        pltpu.sync_copy(m1v, m1_hbm.at[pl.ds(base * 16, n * 16)])
        pltpu.sync_copy(m2v, m2_hbm.at[pl.ds(base * 16, n * 16)])
        pltpu.sync_copy(sv, s_hbm.at[pl.ds(base * 16, n * 16)])

    return sc_stream


def kernel(logits, y):
    b, v = logits.shape
    bt = _BT
    bs = b - bt
    br = 32
    grid = bt // br
    y32 = y.astype(jnp.int32)

    if bt > 0:
        out_tc = pl.pallas_call(
            functools.partial(_tc_body, br=br, v=v),
            grid=(grid,),
            in_specs=[
                pl.BlockSpec(memory_space=pltpu.SMEM),
                pl.BlockSpec((br, v), lambda i: (i, 0)),
            ],
            out_specs=pl.BlockSpec((1, 1, br), lambda i: (i, 0, 0)),
            out_shape=jax.ShapeDtypeStruct((grid, 1, br), jnp.float32),
        )(y32, logits)

    m1l, m2l, sl, segs = _make_sc_stream(b, v, bt)(logits, y32)

    out_sc = pl.pallas_call(
        _sc_finalize_body, out_shape=jax.ShapeDtypeStruct((bs,), jnp.float32)
    )(
        m1l.reshape(bs, 16),
        m2l.reshape(bs, 16),
        sl.reshape(bs, 16),
        segs.reshape(bs, 16),
        y32[bt:].reshape(bs, 1),
    )
    if bt == 0:
        return out_sc
    return jnp.concatenate([out_tc.reshape(bt), out_sc])
